# baseline (speedup 1.0000x reference)
import functools
import math

import jax
import jax.numpy as jnp
from jax import lax
from jax.experimental import pallas as pl
from jax.experimental.pallas import tpu as pltpu
from jax.experimental.pallas import tpu_sc as plsc

_D = 64
_SCALE = float(math.sqrt(_D))
_NC, _NS = 2, 16
_NW = _NC * _NS
_IB = 128
_L = 16
_NBUF = 5


@functools.lru_cache(maxsize=None)
def _make_kernel(R, C):
    n_ti = R // _IB
    n_tk = _D // 8
    n_units = C
    assert n_units % _NBUF == 0 and n_ti == _NW

    mesh = plsc.VectorSubcoreMesh(core_axis_name="c", subcore_axis_name="s")

    @functools.partial(
        pl.kernel,
        out_type=jax.ShapeDtypeStruct((C, n_tk, n_ti, 8, _IB), jnp.float32),
        mesh=mesh,
        compiler_params=pltpu.CompilerParams(
            use_tc_tiling_on_sc=False, needs_layout_passes=False
        ),
        scratch_types=[
            pltpu.VMEM((C, _IB), jnp.int32),
            [pltpu.VMEM((_IB, _D), jnp.float32) for _ in range(_NBUF)],
            [pltpu.VMEM((n_tk, 1, 8, _IB + 1), jnp.float32) for _ in range(_NBUF)],
            [pltpu.SemaphoreType.DMA for _ in range(_NBUF)],
            [pltpu.SemaphoreType.DMA for _ in range(_NBUF)],
        ],
    )
    def emb_kernel(xt_hbm, lut_hbm, out_hbm, idx_v, gbufs, tbufs, gsems, osems):
        wid = lax.axis_index("s") * _NC + lax.axis_index("c")

        pltpu.sync_copy(xt_hbm.at[:, pl.ds(wid * _IB, _IB)], idx_v)

        def fire_gather(u, b):
            pltpu.async_copy(lut_hbm.at[idx_v.at[u]], gbufs[b], gsems[b])

        def drain_gather(b):
            pltpu.make_async_copy(
                lut_hbm.at[pl.ds(0, _IB)], gbufs[b], gsems[b]
            ).wait()

        def out_slice(u):
            return out_hbm.at[u, :, pl.ds(wid, 1)]

        def tbuf_src(b):
            return tbufs[b].at[:, :, :, pl.ds(0, _IB)]

        def wait_out(b):
            pltpu.make_async_copy(tbuf_src(b), out_slice(0), osems[b]).wait()

        for pb in range(_NBUF - 1):
            fire_gather(pb, pb)

        iota = lax.iota(jnp.int32, _L)

        @pl.loop(0, n_units // _NBUF)
        def _(it):
            for b in range(_NBUF):
                u = it * _NBUF + b
                drain_gather(b)

                @pl.when(u >= _NBUF)
                def _():
                    wait_out(b)

                @plsc.parallel_loop(0, _IB, unroll=16)
                def _(i):
                    ivec = jnp.full((_L,), i, jnp.int32)
                    for j in range(_D // _L):
                        cols = iota + j * _L
                        v = gbufs[b][i, pl.ds(j * _L, _L)] * _SCALE
                        tk_v = lax.shift_right_logical(cols, 3)
                        r_v = lax.bitwise_and(cols, jnp.int32(7))
                        plsc.store_scatter(
                            tbufs[b],
                            [tk_v, jnp.zeros((_L,), jnp.int32), r_v, ivec],
                            v,
                        )

                pltpu.async_copy(tbuf_src(b), out_slice(u), osems[b])
                nu = u + (_NBUF - 1)

                @pl.when(nu < n_units)
                def _():
                    fire_gather(nu, (b + _NBUF - 1) % _NBUF)

        for b in range(_NBUF):
            wait_out(b)

    return emb_kernel


def kernel(x, lut):
    R, C = x.shape
    outp = _make_kernel(R, C)(x.T, lut)
    return outp.transpose(2, 4, 0, 1, 3).reshape(R, C, _D)


# speedup vs baseline: 1.0152x; 1.0152x over previous
import functools
import math

import jax
import jax.numpy as jnp
from jax import lax
from jax.experimental import pallas as pl
from jax.experimental.pallas import tpu as pltpu
from jax.experimental.pallas import tpu_sc as plsc

_D = 64
_SCALE = float(math.sqrt(_D))
_NC, _NS = 2, 16
_NW = _NC * _NS
_IB = 128
_L = 16
_NBUF = 5


@functools.lru_cache(maxsize=None)
def _make_kernel(R, C):
    n_ti = R // _IB
    n_tk = _D // 8
    n_units = C
    assert n_units % _NBUF == 0 and n_ti == _NW

    mesh = plsc.VectorSubcoreMesh(core_axis_name="c", subcore_axis_name="s")

    @functools.partial(
        pl.kernel,
        out_type=jax.ShapeDtypeStruct((C, n_tk, n_ti, 8, _IB), jnp.float32),
        mesh=mesh,
        compiler_params=pltpu.CompilerParams(
            use_tc_tiling_on_sc=False, needs_layout_passes=False
        ),
        scratch_types=[
            pltpu.VMEM((C, _IB), jnp.int32),
            [pltpu.VMEM((_IB, _D), jnp.float32) for _ in range(_NBUF)],
            [pltpu.VMEM((n_tk, 1, 8, _IB + 1), jnp.float32) for _ in range(_NBUF)],
            [pltpu.SemaphoreType.DMA for _ in range(_NBUF)],
            [pltpu.SemaphoreType.DMA for _ in range(_NBUF)],
        ],
    )
    def emb_kernel(xt_hbm, lut_hbm, out_hbm, idx_v, gbufs, tbufs, gsems, osems):
        wid = lax.axis_index("s") * _NC + lax.axis_index("c")

        pltpu.sync_copy(xt_hbm.at[:, pl.ds(wid * _IB, _IB)], idx_v)

        def fire_gather(u, b):
            pltpu.async_copy(lut_hbm.at[idx_v.at[u]], gbufs[b], gsems[b])

        def drain_gather(b):
            pltpu.make_async_copy(
                lut_hbm.at[pl.ds(0, _IB)], gbufs[b], gsems[b]
            ).wait()

        def out_slice(u):
            return out_hbm.at[u, :, pl.ds(wid, 1)]

        def tbuf_src(b):
            return tbufs[b].at[:, :, :, pl.ds(0, _IB)]

        def wait_out(b):
            pltpu.make_async_copy(tbuf_src(b), out_slice(0), osems[b]).wait()

        for pb in range(_NBUF - 1):
            fire_gather(pb, pb)

        iota = lax.iota(jnp.int32, _L)

        @pl.loop(0, n_units // _NBUF)
        def _(it):
            for b in range(_NBUF):
                u = it * _NBUF + b
                drain_gather(b)

                @pl.when(u >= _NBUF)
                def _():
                    wait_out(b)

                @plsc.parallel_loop(0, _IB, unroll=8)
                def _(i):
                    ivec = jnp.full((_L,), i, jnp.int32)
                    for j in range(_D // _L):
                        cols = iota + j * _L
                        v = gbufs[b][i, pl.ds(j * _L, _L)] * _SCALE
                        tk_v = lax.shift_right_logical(cols, 3)
                        r_v = lax.bitwise_and(cols, jnp.int32(7))
                        plsc.store_scatter(
                            tbufs[b],
                            [tk_v, jnp.zeros((_L,), jnp.int32), r_v, ivec],
                            v,
                        )

                pltpu.async_copy(tbuf_src(b), out_slice(u), osems[b])
                nu = u + (_NBUF - 1)

                @pl.when(nu < n_units)
                def _():
                    fire_gather(nu, (b + _NBUF - 1) % _NBUF)

        for b in range(_NBUF):
            wait_out(b)

    return emb_kernel


def kernel(x, lut):
    R, C = x.shape
    outp = _make_kernel(R, C)(x.T, lut)
    return outp.transpose(2, 4, 0, 1, 3).reshape(R, C, _D)
